# speedup vs baseline: 2.2678x; 2.2678x over previous
import functools

import jax
import jax.numpy as jnp
from jax import lax
from jax.experimental import pallas as pl
from jax.experimental.pallas import tpu as pltpu
from jax.experimental.pallas import tpu_sc as plsc


def kernel(style_ids, frame_ids, latents, latents_mu):
    S, F, D = latents.shape
    B = style_ids.shape[0]
    flat_table = latents.reshape(S * F, D)
    style_ids = style_ids.astype(jnp.int32)
    frame_ids = frame_ids.astype(jnp.int32)

    info = plsc.get_sparse_core_info()
    NC, NS, L = info.num_cores, info.num_subcores, info.num_lanes
    NW = NC * NS
    b_per_w = B // NW
    H = b_per_w // 2

    mesh = plsc.VectorSubcoreMesh(core_axis_name="c", subcore_axis_name="s")

    @functools.partial(
        pl.kernel,
        mesh=mesh,
        out_type=jax.ShapeDtypeStruct((B, D), jnp.float32),
        scratch_types=[
            pltpu.VMEM((b_per_w,), jnp.int32),
            pltpu.VMEM((b_per_w,), jnp.int32),
            pltpu.VMEM((H, D), jnp.float32),
            pltpu.VMEM((H, D), jnp.float32),
            pltpu.SemaphoreType.DMA,
            pltpu.SemaphoreType.DMA,
        ],
    )
    def run(style_hbm, frame_hbm, table_hbm, mu_hbm, out_hbm,
            sty_v, idx_v, rows_v, mu_v, sem_a, sem_b):
        wid = lax.axis_index("s") * NC + lax.axis_index("c")
        base = wid * b_per_w
        pltpu.sync_copy(style_hbm.at[pl.ds(base, b_per_w)], sty_v)
        pltpu.sync_copy(frame_hbm.at[pl.ds(base, b_per_w)], idx_v)

        def flat_body(i, _):
            sl = pl.ds(i * L, L)
            idx_v[sl] = sty_v[sl] * F + idx_v[sl]
            return 0
        lax.fori_loop(0, b_per_w // L, flat_body, 0)

        for h in range(2):
            hbase = h * H
            cp_rows = pltpu.async_copy(
                table_hbm.at[idx_v.at[pl.ds(hbase, H)]], rows_v, sem_a)
            cp_mu = pltpu.async_copy(
                mu_hbm.at[sty_v.at[pl.ds(hbase, H)]], mu_v, sem_b)
            cp_rows.wait()
            cp_mu.wait()

            def add_body(i, _):
                for j in range(D // L):
                    sl = pl.ds(j * L, L)
                    rows_v[i, sl] = rows_v[i, sl] + mu_v[i, sl]
                return 0
            lax.fori_loop(0, H, add_body, 0)

            pltpu.sync_copy(rows_v, out_hbm.at[pl.ds(base + hbase, H)])

    return run(style_ids, frame_ids, flat_table, latents_mu)


# speedup vs baseline: 2.4448x; 1.0781x over previous
import functools

import jax
import jax.numpy as jnp
from jax import lax
from jax.experimental import pallas as pl
from jax.experimental.pallas import tpu as pltpu
from jax.experimental.pallas import tpu_sc as plsc


def kernel(style_ids, frame_ids, latents, latents_mu):
    S, F, D = latents.shape
    B = style_ids.shape[0]
    flat_table = latents.reshape(S * F, D)
    style_ids = style_ids.astype(jnp.int32)
    frame_ids = frame_ids.astype(jnp.int32)

    info = plsc.get_sparse_core_info()
    NC, NS, L = info.num_cores, info.num_subcores, info.num_lanes
    NW = NC * NS
    b_per_w = B // NW
    H = b_per_w // 2

    mesh = plsc.VectorSubcoreMesh(core_axis_name="c", subcore_axis_name="s")

    @functools.partial(
        pl.kernel,
        mesh=mesh,
        out_type=jax.ShapeDtypeStruct((B, D), jnp.float32),
        scratch_types=[
            pltpu.VMEM((b_per_w,), jnp.int32),
            pltpu.VMEM((b_per_w,), jnp.int32),
            pltpu.VMEM((H, D), jnp.float32),
            pltpu.VMEM((H, D), jnp.float32),
            pltpu.SemaphoreType.DMA,
            pltpu.SemaphoreType.DMA,
            pltpu.SemaphoreType.DMA,
        ],
    )
    def run(style_hbm, frame_hbm, table_hbm, mu_hbm, out_hbm,
            sty_v, idx_v, rows0, rows1, sem_a, sem_b, sem_w):
        wid = lax.axis_index("s") * NC + lax.axis_index("c")
        base = wid * b_per_w
        pltpu.sync_copy(style_hbm.at[pl.ds(base, b_per_w)], sty_v)
        pltpu.sync_copy(frame_hbm.at[pl.ds(base, b_per_w)], idx_v)

        def flat_body(i, _):
            sl = pl.ds(i * L, L)
            idx_v[sl] = sty_v[sl] * F + idx_v[sl]
            return 0
        lax.fori_loop(0, b_per_w // L, flat_body, 0)

        g0 = pltpu.async_copy(table_hbm.at[idx_v.at[pl.ds(0, H)]], rows0, sem_a)
        g0.wait()
        a0 = pltpu.async_copy(mu_hbm.at[sty_v.at[pl.ds(0, H)]], rows0, sem_a,
                              add=True)
        g1 = pltpu.async_copy(table_hbm.at[idx_v.at[pl.ds(H, H)]], rows1, sem_b)
        a0.wait()
        w0 = pltpu.async_copy(rows0, out_hbm.at[pl.ds(base, H)], sem_w)
        g1.wait()
        a1 = pltpu.async_copy(mu_hbm.at[sty_v.at[pl.ds(H, H)]], rows1, sem_b,
                              add=True)
        a1.wait()
        w0.wait()
        pltpu.sync_copy(rows1, out_hbm.at[pl.ds(base + H, H)])

    return run(style_ids, frame_ids, flat_table, latents_mu)


# speedup vs baseline: 2.5332x; 1.0362x over previous
import functools

import jax
import jax.numpy as jnp
from jax import lax
from jax.experimental import pallas as pl
from jax.experimental.pallas import tpu as pltpu
from jax.experimental.pallas import tpu_sc as plsc


def kernel(style_ids, frame_ids, latents, latents_mu):
    S, F, D = latents.shape
    B = style_ids.shape[0]
    flat_table = latents.reshape(S * F, D)
    style_ids = style_ids.astype(jnp.int32)
    frame_ids = frame_ids.astype(jnp.int32)

    info = plsc.get_sparse_core_info()
    NC, NS, L = info.num_cores, info.num_subcores, info.num_lanes
    NW = NC * NS
    b_per_w = B // NW
    H = b_per_w // 2

    mesh = plsc.VectorSubcoreMesh(core_axis_name="c", subcore_axis_name="s")

    @functools.partial(
        pl.kernel,
        mesh=mesh,
        out_type=jax.ShapeDtypeStruct((B, D), jnp.float32),
        scratch_types=[
            pltpu.VMEM((b_per_w,), jnp.int32),
            pltpu.VMEM((b_per_w,), jnp.int32),
            pltpu.VMEM((b_per_w, D), jnp.float32),
            pltpu.SemaphoreType.DMA,
            pltpu.SemaphoreType.DMA,
        ],
    )
    def run(style_hbm, frame_hbm, table_hbm, mu_hbm, out_hbm,
            sty_v, idx_v, rows_v, sem_a, sem_b):
        wid = lax.axis_index("s") * NC + lax.axis_index("c")
        base = wid * b_per_w
        c_sty = pltpu.async_copy(style_hbm.at[pl.ds(base, b_per_w)], sty_v,
                                 sem_a)
        c_frm = pltpu.async_copy(frame_hbm.at[pl.ds(base, b_per_w)], idx_v,
                                 sem_b)
        c_sty.wait()
        c_frm.wait()

        def flat_body(i, _):
            sl = pl.ds(i * L, L)
            idx_v[sl] = sty_v[sl] * F + idx_v[sl]
            return 0
        lax.fori_loop(0, b_per_w // L, flat_body, 0)

        pltpu.async_copy(table_hbm.at[idx_v], rows_v, sem_a).wait()
        pltpu.async_copy(mu_hbm.at[sty_v], rows_v, sem_a, add=True).wait()
        pltpu.sync_copy(rows_v, out_hbm.at[pl.ds(base, b_per_w)])

    return run(style_ids, frame_ids, flat_table, latents_mu)
